# baseline (speedup 1.0000x reference)
import functools

import jax
import jax.numpy as jnp
from jax import lax
from jax.experimental import pallas as pl
from jax.experimental.pallas import tpu as pltpu
from jax.experimental.pallas import tpu_sc as plsc

N = 10000
D = 128
NT = 16
NW = 32
EB = 128
RPT = 640
RPL = N - 15 * RPT
RCH = 40
SBLK = 8
CB0 = 160
CB1 = 0
MB = 1000


def _mm_body(x_ref, k_ref, o_ref):
    o_ref[...] = jnp.dot(x_ref[...], k_ref[...],
                         preferred_element_type=jnp.float32)


_matmul = pl.pallas_call(
    _mm_body,
    grid=(N // MB,),
    in_specs=[
        pl.BlockSpec((MB, D), lambda i: (i, 0)),
        pl.BlockSpec((D, D), lambda i: (0, 0)),
    ],
    out_specs=pl.BlockSpec((MB, D), lambda i: (i, 0)),
    out_shape=jax.ShapeDtypeStruct((N, D), jnp.float32),
)


def _comb_filt_body(p_ref, f_ref, o_ref):
    o_ref[...] = (p_ref[0] + p_ref[1]) * f_ref[...]


_combine_filt = pl.pallas_call(
    _comb_filt_body,
    grid=(N // MB,),
    in_specs=[
        pl.BlockSpec((2, MB, D), lambda i: (0, i, 0)),
        pl.BlockSpec((MB, 1), lambda i: (i, 0)),
    ],
    out_specs=pl.BlockSpec((MB, D), lambda i: (i, 0)),
    out_shape=jax.ShapeDtypeStruct((N, D), jnp.float32),
)


def _comb_body(p_ref, o_ref):
    o_ref[...] = p_ref[0] + p_ref[1]


_combine = pl.pallas_call(
    _comb_body,
    grid=(N // MB,),
    in_specs=[pl.BlockSpec((2, MB, D), lambda i: (0, i, 0))],
    out_specs=pl.BlockSpec((MB, D), lambda i: (i, 0)),
    out_shape=jax.ShapeDtypeStruct((N, D), jnp.float32),
)


_BCAST_DN = lax.GatherDimensionNumbers(
    offset_dims=(), collapsed_slice_dims=(0,), start_index_map=(0,))


def _bcast_lane(vv, u):
    return lax.gather(vv, jnp.full((16, 1), u, jnp.int32), _BCAST_DN, (1,),
                      mode=lax.GatherScatterMode.PROMISE_IN_BOUNDS)


@functools.lru_cache(maxsize=None)
def _make_spmm(nbt: int):
    assert nbt == NT * (CB0 + CB1)
    mesh = plsc.VectorSubcoreMesh(core_axis_name="c", subcore_axis_name="s")
    scratch = [
        pltpu.VMEM_SHARED((N, D), jnp.float32),
        pltpu.VMEM((SBLK, EB), jnp.int32),
        pltpu.VMEM((SBLK, EB), jnp.int32),
        pltpu.VMEM((SBLK, EB), jnp.float32),
        pltpu.VMEM((EB, D), jnp.float32),
        pltpu.VMEM((EB, D), jnp.float32),
        pltpu.VMEM((RCH, D), jnp.float32),
        pltpu.SemaphoreType.DMA,
        pltpu.SemaphoreType.DMA,
    ]

    def body(h, colsp, rowsp, valsp, out,
             acc, colsb, rowsb, valsb, gbuf_a, gbuf_b, obuf, sem_a, sem_b):
        c = lax.axis_index("c")
        s = lax.axis_index("s")
        base = s * RPT
        nrow = jnp.where(s == NT - 1, RPL, RPT)

        def zb(i, carry):
            for f in range(D // 16):
                obuf[i, pl.ds(f * 16, 16)] = jnp.zeros((16,), jnp.float32)
            return carry
        lax.fori_loop(0, RCH, zb, 0)

        nch = nrow // RCH
        for k2 in range(RPT // RCH):
            @pl.when(k2 < nch)
            def _():
                pltpu.async_copy(obuf, acc.at[pl.ds(base + k2 * RCH, RCH)],
                                 sem_a)
        for k2 in range(RPT // RCH):
            @pl.when(k2 < nch)
            def _():
                pltpu.make_async_copy(
                    obuf, acc.at[pl.ds(base + k2 * RCH, RCH)], sem_a).wait()

        bstart = jnp.where(c == 0, s * CB0, NT * CB0 + s * CB1)
        nb = jnp.where(c == 0, CB0, CB1)
        plsc.subcore_barrier()

        def scale(buf, j):
            def se(g, c2):
                vv = valsb[j, pl.ds(g * 16, 16)]
                for u in range(16):
                    i = g * 16 + u
                    v = jnp.full((16,), vv[u], jnp.float32)
                    for f in range(D // 16):
                        sl = pl.ds(f * 16, 16)
                        buf[i, sl] = buf[i, sl] * v
                return c2
            lax.fori_loop(0, EB // 16, se, 0)

        def sb_loop(sb, carry):
            sb0 = bstart + sb * SBLK
            cp_c = pltpu.async_copy(colsp.at[pl.ds(sb0, SBLK)], colsb, sem_a)
            cp_r = pltpu.async_copy(rowsp.at[pl.ds(sb0, SBLK)], rowsb, sem_a)
            cp_v = pltpu.async_copy(valsp.at[pl.ds(sb0, SBLK)], valsb, sem_a)
            cp_c.wait()
            cp_r.wait()
            cp_v.wait()

            pltpu.async_copy(h.at[colsb.at[0]], gbuf_a, sem_a)

            def blk2(j2, c1):
                ja = 2 * j2
                jb = 2 * j2 + 1
                pltpu.make_async_copy(h.at[colsb.at[ja]], gbuf_a,
                                      sem_a).wait()
                pltpu.async_copy(h.at[colsb.at[jb]], gbuf_b, sem_b)
                scale(gbuf_a, ja)
                pltpu.sync_copy(gbuf_a, acc.at[rowsb.at[ja]], add=True)

                pltpu.make_async_copy(h.at[colsb.at[jb]], gbuf_b,
                                      sem_b).wait()

                @pl.when(j2 < SBLK // 2 - 1)
                def _():
                    pltpu.async_copy(h.at[colsb.at[ja + 2]], gbuf_a, sem_a)

                scale(gbuf_b, jb)
                pltpu.sync_copy(gbuf_b, acc.at[rowsb.at[jb]], add=True)
                return c1
            lax.fori_loop(0, SBLK // 2, blk2, 0)
            return carry
        lax.fori_loop(0, nb // SBLK, sb_loop, 0)

        plsc.subcore_barrier()

        @pl.when(s < NT - 1)
        def _():
            pltpu.sync_copy(acc.at[pl.ds(base, RPT)],
                            out.at[c].at[pl.ds(base, RPT)])

        @pl.when(s == NT - 1)
        def _():
            pltpu.sync_copy(acc.at[pl.ds(base, RPL)],
                            out.at[c].at[pl.ds(base, RPL)])

    return pl.kernel(
        body,
        out_type=jax.ShapeDtypeStruct((2, N, D), jnp.float32),
        mesh=mesh,
        scratch_types=scratch,
    )


def _prep_edges(rows, cols, vals):
    e = rows.shape[0]
    nbt = NT * (CB0 + CB1)
    tot = nbt * EB
    pad = tot - e
    assert pad >= 0
    rows_p = jnp.concatenate([rows, jnp.zeros((pad,), rows.dtype)])
    cols_p = jnp.concatenate([cols, jnp.zeros((pad,), cols.dtype)])
    vals_p = jnp.concatenate([vals, jnp.zeros((pad,), vals.dtype)])
    shape = (nbt, EB)
    return (nbt, cols_p.reshape(shape), rows_p.reshape(shape),
            vals_p.reshape(shape))


def kernel(x, wavelet_indices, wavelet_values, inverse_wavelet_indices,
           inverse_wavelet_values, kernel, filt):
    h = _matmul(x, kernel)

    nblk1, colsp1, rowsp1, valsp1 = _prep_edges(
        inverse_wavelet_indices[0], inverse_wavelet_indices[1],
        inverse_wavelet_values)
    p1 = _make_spmm(nblk1)(h, colsp1, rowsp1, valsp1)
    h1 = _combine_filt(p1, filt.reshape(N, 1))

    nblk2, colsp2, rowsp2, valsp2 = _prep_edges(
        wavelet_indices[0], wavelet_indices[1], wavelet_values)
    p2 = _make_spmm(nblk2)(h1, colsp2, rowsp2, valsp2)
    return _combine(p2)


# speedup vs baseline: 1.5597x; 1.5597x over previous
import functools

import jax
import jax.numpy as jnp
from jax import lax
from jax.experimental import pallas as pl
from jax.experimental.pallas import tpu as pltpu
from jax.experimental.pallas import tpu_sc as plsc

N = 10000
D = 128
NT = 16
NW = 32
EB = 128
RPT = 640
RPL = N - 15 * RPT
RCH = 40
SBLK = 8
CB0 = 144
CB1 = 16
MB = 1000


def _mm_body(x_ref, k_ref, o_ref):
    o_ref[...] = jnp.dot(x_ref[...], k_ref[...],
                         preferred_element_type=jnp.float32)


_matmul = pl.pallas_call(
    _mm_body,
    grid=(N // MB,),
    in_specs=[
        pl.BlockSpec((MB, D), lambda i: (i, 0)),
        pl.BlockSpec((D, D), lambda i: (0, 0)),
    ],
    out_specs=pl.BlockSpec((MB, D), lambda i: (i, 0)),
    out_shape=jax.ShapeDtypeStruct((N, D), jnp.float32),
)


def _comb_filt_body(p_ref, f_ref, o_ref):
    o_ref[...] = (p_ref[0] + p_ref[1]) * f_ref[...]


_combine_filt = pl.pallas_call(
    _comb_filt_body,
    grid=(N // MB,),
    in_specs=[
        pl.BlockSpec((2, MB, D), lambda i: (0, i, 0)),
        pl.BlockSpec((MB, 1), lambda i: (i, 0)),
    ],
    out_specs=pl.BlockSpec((MB, D), lambda i: (i, 0)),
    out_shape=jax.ShapeDtypeStruct((N, D), jnp.float32),
)


def _comb_body(p_ref, o_ref):
    o_ref[...] = p_ref[0] + p_ref[1]


_combine = pl.pallas_call(
    _comb_body,
    grid=(N // MB,),
    in_specs=[pl.BlockSpec((2, MB, D), lambda i: (0, i, 0))],
    out_specs=pl.BlockSpec((MB, D), lambda i: (i, 0)),
    out_shape=jax.ShapeDtypeStruct((N, D), jnp.float32),
)


_BCAST_DN = lax.GatherDimensionNumbers(
    offset_dims=(), collapsed_slice_dims=(0,), start_index_map=(0,))


def _bcast_lane(vv, u):
    return lax.gather(vv, jnp.full((16, 1), u, jnp.int32), _BCAST_DN, (1,),
                      mode=lax.GatherScatterMode.PROMISE_IN_BOUNDS)


@functools.lru_cache(maxsize=None)
def _make_spmm(nbt: int):
    assert nbt == NT * (CB0 + CB1)
    mesh = plsc.VectorSubcoreMesh(core_axis_name="c", subcore_axis_name="s")
    scratch = [
        pltpu.VMEM_SHARED((N, D), jnp.float32),
        pltpu.VMEM((SBLK, EB), jnp.int32),
        pltpu.VMEM((SBLK, EB), jnp.int32),
        pltpu.VMEM((SBLK, EB), jnp.float32),
        pltpu.VMEM((EB, D), jnp.float32),
        pltpu.VMEM((EB, D), jnp.float32),
        pltpu.VMEM((RCH, D), jnp.float32),
        pltpu.SemaphoreType.DMA,
        pltpu.SemaphoreType.DMA,
    ]

    def body(h, colsp, rowsp, valsp, out,
             acc, colsb, rowsb, valsb, gbuf_a, gbuf_b, obuf, sem_a, sem_b):
        c = lax.axis_index("c")
        s = lax.axis_index("s")
        base = s * RPT
        nrow = jnp.where(s == NT - 1, RPL, RPT)

        def zb(i, carry):
            for f in range(D // 16):
                obuf[i, pl.ds(f * 16, 16)] = jnp.zeros((16,), jnp.float32)
            return carry
        lax.fori_loop(0, RCH, zb, 0)

        nch = nrow // RCH
        for k2 in range(RPT // RCH):
            @pl.when(k2 < nch)
            def _():
                pltpu.async_copy(obuf, acc.at[pl.ds(base + k2 * RCH, RCH)],
                                 sem_a)
        for k2 in range(RPT // RCH):
            @pl.when(k2 < nch)
            def _():
                pltpu.make_async_copy(
                    obuf, acc.at[pl.ds(base + k2 * RCH, RCH)], sem_a).wait()

        bstart = jnp.where(c == 0, s * CB0, NT * CB0 + s * CB1)
        nb = jnp.where(c == 0, CB0, CB1)
        plsc.subcore_barrier()

        def scale(buf, j):
            def se(g, c2):
                vv = valsb[j, pl.ds(g * 16, 16)]
                for u in range(16):
                    i = g * 16 + u
                    v = jnp.full((16,), vv[u], jnp.float32)
                    for f in range(D // 16):
                        sl = pl.ds(f * 16, 16)
                        buf[i, sl] = buf[i, sl] * v
                return c2
            lax.fori_loop(0, EB // 16, se, 0)

        def sb_loop(sb, carry):
            sb0 = bstart + sb * SBLK
            cp_c = pltpu.async_copy(colsp.at[pl.ds(sb0, SBLK)], colsb, sem_a)
            cp_r = pltpu.async_copy(rowsp.at[pl.ds(sb0, SBLK)], rowsb, sem_a)
            cp_v = pltpu.async_copy(valsp.at[pl.ds(sb0, SBLK)], valsb, sem_a)
            cp_c.wait()
            cp_r.wait()
            cp_v.wait()

            pltpu.async_copy(h.at[colsb.at[0]], gbuf_a, sem_a)

            def blk2(j2, c1):
                ja = 2 * j2
                jb = 2 * j2 + 1
                pltpu.make_async_copy(h.at[colsb.at[ja]], gbuf_a,
                                      sem_a).wait()
                pltpu.async_copy(h.at[colsb.at[jb]], gbuf_b, sem_b)
                scale(gbuf_a, ja)
                pltpu.sync_copy(gbuf_a, acc.at[rowsb.at[ja]], add=True)

                pltpu.make_async_copy(h.at[colsb.at[jb]], gbuf_b,
                                      sem_b).wait()

                @pl.when(j2 < SBLK // 2 - 1)
                def _():
                    pltpu.async_copy(h.at[colsb.at[ja + 2]], gbuf_a, sem_a)

                scale(gbuf_b, jb)
                pltpu.sync_copy(gbuf_b, acc.at[rowsb.at[jb]], add=True)
                return c1
            lax.fori_loop(0, SBLK // 2, blk2, 0)
            return carry
        lax.fori_loop(0, nb // SBLK, sb_loop, 0)

        plsc.subcore_barrier()

        @pl.when(s < NT - 1)
        def _():
            pltpu.sync_copy(acc.at[pl.ds(base, RPT)],
                            out.at[c].at[pl.ds(base, RPT)])

        @pl.when(s == NT - 1)
        def _():
            pltpu.sync_copy(acc.at[pl.ds(base, RPL)],
                            out.at[c].at[pl.ds(base, RPL)])

    return pl.kernel(
        body,
        out_type=jax.ShapeDtypeStruct((2, N, D), jnp.float32),
        mesh=mesh,
        scratch_types=scratch,
    )


def _prep_edges(rows, cols, vals):
    e = rows.shape[0]
    nbt = NT * (CB0 + CB1)
    tot = nbt * EB
    pad = tot - e
    assert pad >= 0
    rows_p = jnp.concatenate([rows, jnp.zeros((pad,), rows.dtype)])
    cols_p = jnp.concatenate([cols, jnp.zeros((pad,), cols.dtype)])
    vals_p = jnp.concatenate([vals, jnp.zeros((pad,), vals.dtype)])
    shape = (nbt, EB)
    return (nbt, cols_p.reshape(shape), rows_p.reshape(shape),
            vals_p.reshape(shape))


def kernel(x, wavelet_indices, wavelet_values, inverse_wavelet_indices,
           inverse_wavelet_values, kernel, filt):
    h = _matmul(x, kernel)

    nblk1, colsp1, rowsp1, valsp1 = _prep_edges(
        inverse_wavelet_indices[0], inverse_wavelet_indices[1],
        inverse_wavelet_values)
    p1 = _make_spmm(nblk1)(h, colsp1, rowsp1, valsp1)
    h1 = _combine_filt(p1, filt.reshape(N, 1))

    nblk2, colsp2, rowsp2, valsp2 = _prep_edges(
        wavelet_indices[0], wavelet_indices[1], wavelet_values)
    p2 = _make_spmm(nblk2)(h1, colsp2, rowsp2, valsp2)
    return _combine(p2)
